# Initial kernel scaffold; baseline (speedup 1.0000x reference)
import jax
import jax.numpy as jnp
from jax.experimental import pallas as pl


def kernel(sentence, gazet, char_table):
    raise NotImplementedError("write your pallas kernel here")


# speedup vs baseline: 1.4592x; 1.4592x over previous
import jax
import jax.numpy as jnp
from jax import lax
from jax.experimental import pallas as pl
from jax.experimental.pallas import tpu as pltpu
from jax.experimental.pallas import tpu_sc as plsc

VOCAB = 1000
CHAR_DIM = 40
GAZET_DIM = 15
CTX = 21
SEQ = 16384
EMBED = CHAR_DIM + GAZET_DIM
PAD_D = 64

NC, NS = 2, 16
NW = NC * NS
ROWS_W = SEQ // NW
R_BLK = 32
N_BLK = ROWS_W // R_BLK
TOK_BLK = R_BLK * CTX
G_CHUNK = 96
N_CHUNK = TOK_BLK // G_CHUNK
GAZ_BLK = TOK_BLK * GAZET_DIM
TG = 16
N_GRP = TOK_BLK // TG
GRP_W = TG * EMBED
N_SLC = GRP_W // 16


def _pe_table():
    j = jnp.arange(1, CTX + 1, dtype=jnp.float32)[:, None]
    k = jnp.arange(1, EMBED + 1, dtype=jnp.float32)[None, :]
    return 1.0 - j / CTX - k / EMBED * (1.0 - 2.0 * j / CTX)


def _sc_body(aug_hbm, idx_hbm, gaz_hbm, rowt_hbm, colt_hbm, out_hbm,
             idx_v, gaz_v, stg_v, out_v, rowt_v, colt_v, sem):
    wid = lax.axis_index("s") * NC + lax.axis_index("c")

    pltpu.sync_copy(rowt_hbm, rowt_v)
    pltpu.sync_copy(colt_hbm, colt_v)

    def block(i, carry):
        tok0 = (wid * ROWS_W + i * R_BLK) * CTX

        pltpu.sync_copy(idx_hbm.at[pl.ds(tok0, TOK_BLK)], idx_v)

        copies = []
        for cch in range(N_CHUNK):
            copies.append(
                pltpu.async_copy(
                    aug_hbm.at[idx_v.at[pl.ds(cch * G_CHUNK, G_CHUNK)]],
                    stg_v.at[pl.ds(cch * G_CHUNK, G_CHUNK)],
                    sem,
                )
            )
        pltpu.sync_copy(
            gaz_hbm.at[pl.ds(tok0 * GAZET_DIM, GAZ_BLK)],
            gaz_v.at[pl.ds(0, GAZ_BLK)],
        )
        for c in copies:
            c.wait()

        lanes = lax.iota(jnp.int32, 16)
        mask = lanes < GAZET_DIM
        col_idx = lanes + CHAR_DIM

        def tok(t, carry):
            g = plsc.load_gather(gaz_v, [t * GAZET_DIM + lanes])
            row_idx = jnp.full((16,), t, jnp.int32)
            plsc.addupdate_scatter(stg_v, [row_idx, col_idx], g, mask=mask)
            return carry

        lax.fori_loop(0, TOK_BLK, tok, 0, unroll=4)

        def grp(g, carry):
            row0 = jnp.full((16,), g * TG, jnp.int32)
            for j in range(N_SLC):
                rows = rowt_v[j] + row0
                vals = plsc.load_gather(stg_v, [rows, colt_v[j]])
                out_v[pl.ds(g * GRP_W + j * 16, 16)] = vals
            return carry

        lax.fori_loop(0, N_GRP, grp, 0)

        pltpu.sync_copy(
            out_v, out_hbm.at[pl.ds(tok0 * EMBED, TOK_BLK * EMBED)]
        )
        return carry

    lax.fori_loop(0, N_BLK, block, 0)


@jax.jit
def _run(aug, idx_flat, gaz_flat, rowt, colt):
    mesh = plsc.VectorSubcoreMesh(core_axis_name="c", subcore_axis_name="s")
    k = pl.kernel(
        _sc_body,
        out_type=jax.ShapeDtypeStruct((SEQ * CTX * EMBED,), jnp.float32),
        mesh=mesh,
        scratch_types=[
            pltpu.VMEM((TOK_BLK,), jnp.int32),
            pltpu.VMEM((GAZ_BLK + 16,), jnp.float32),
            pltpu.VMEM((TOK_BLK, PAD_D), jnp.float32),
            pltpu.VMEM((TOK_BLK * EMBED,), jnp.float32),
            pltpu.VMEM((N_SLC, 16), jnp.int32),
            pltpu.VMEM((N_SLC, 16), jnp.int32),
            pltpu.SemaphoreType.DMA,
        ],
        compiler_params=pltpu.CompilerParams(
            needs_layout_passes=False, use_tc_tiling_on_sc=False
        ),
    )
    return k(aug, idx_flat, gaz_flat, rowt, colt)


def kernel(sentence, gazet, char_table):
    pe = _pe_table()
    aug = jnp.concatenate(
        [
            char_table[None, :, :] + pe[:, None, :CHAR_DIM],
            jnp.broadcast_to(pe[:, None, CHAR_DIM:], (CTX, VOCAB, GAZET_DIM)),
            jnp.zeros((CTX, VOCAB, PAD_D - EMBED), jnp.float32),
        ],
        axis=2,
    ).reshape(CTX * VOCAB, PAD_D)
    idx_flat = (
        sentence.astype(jnp.int32) + jnp.arange(CTX, dtype=jnp.int32)[None, :] * VOCAB
    ).reshape(SEQ * CTX)
    gaz_flat = gazet.reshape(SEQ * CTX * GAZET_DIM)
    p = jnp.arange(GRP_W, dtype=jnp.int32).reshape(N_SLC, 16)
    rowt = p // EMBED
    colt = p % EMBED
    out = _run(aug, idx_flat, gaz_flat, rowt, colt)
    return out.reshape(SEQ, CTX, EMBED)


# speedup vs baseline: 5.2662x; 3.6090x over previous
import jax
import jax.numpy as jnp
from jax import lax
from jax.experimental import pallas as pl
from jax.experimental.pallas import tpu as pltpu
from jax.experimental.pallas import tpu_sc as plsc

VOCAB = 1000
CHAR_DIM = 40
GAZET_DIM = 15
CTX = 21
SEQ = 16384
EMBED = CHAR_DIM + GAZET_DIM

NC, NS = 2, 16
NW = NC * NS
S_W = SEQ // NW
N_SL = S_W // 16


def _sc_body(tab_hbm, sent_hbm, gaz_hbm, out_hbm, tab_v, idx_v, gaz_v, out_v):
    wid = lax.axis_index("s") * NC + lax.axis_index("c")
    s0 = wid * S_W

    pltpu.sync_copy(tab_hbm, tab_v)
    pltpu.sync_copy(sent_hbm.at[:, pl.ds(s0, S_W)], idx_v)

    def ctx(c, carry):
        pltpu.sync_copy(gaz_hbm.at[c, :, pl.ds(s0, S_W)], gaz_v)
        jc = (c.astype(jnp.float32) + 1.0) * (1.0 / CTX)

        def chunk(j, carry):
            idx = idx_v[c, pl.ds(j * 16, 16)]
            for e in range(CHAR_DIM):
                ke = float(e + 1) / EMBED
                pe_ce = (1.0 - ke) - jc * (1.0 - 2.0 * ke)
                vals = plsc.load_gather(tab_v, [idx + e * VOCAB])
                out_v[e, pl.ds(j * 16, 16)] = vals + pe_ce
            for g in range(GAZET_DIM):
                e = CHAR_DIM + g
                ke = float(e + 1) / EMBED
                pe_ce = (1.0 - ke) - jc * (1.0 - 2.0 * ke)
                gv = gaz_v[g, pl.ds(j * 16, 16)]
                out_v[e, pl.ds(j * 16, 16)] = gv + pe_ce
            return carry

        lax.fori_loop(0, N_SL, chunk, 0)
        pltpu.sync_copy(out_v, out_hbm.at[c, :, pl.ds(s0, S_W)])
        return carry

    lax.fori_loop(0, CTX, ctx, 0)


@jax.jit
def _run(tab_flat, sent_t, gaz_t):
    mesh = plsc.VectorSubcoreMesh(core_axis_name="c", subcore_axis_name="s")
    k = pl.kernel(
        _sc_body,
        out_type=jax.ShapeDtypeStruct((CTX, EMBED, SEQ), jnp.float32),
        mesh=mesh,
        scratch_types=[
            pltpu.VMEM((CHAR_DIM * VOCAB,), jnp.float32),
            pltpu.VMEM((CTX, S_W), jnp.int32),
            pltpu.VMEM((GAZET_DIM, S_W), jnp.float32),
            pltpu.VMEM((EMBED, S_W), jnp.float32),
        ],
        compiler_params=pltpu.CompilerParams(
            needs_layout_passes=False, use_tc_tiling_on_sc=False
        ),
    )
    return k(tab_flat, sent_t, gaz_t)


def kernel(sentence, gazet, char_table):
    tab_flat = char_table.T.reshape(CHAR_DIM * VOCAB)
    sent_t = jnp.swapaxes(sentence, 0, 1).astype(jnp.int32)
    gaz_t = jnp.transpose(gazet, (1, 2, 0))
    out = _run(tab_flat, sent_t, gaz_t)
    return jnp.transpose(out, (2, 0, 1))


# speedup vs baseline: 6.1946x; 1.1763x over previous
import jax
import jax.numpy as jnp
from jax import lax
from jax.experimental import pallas as pl
from jax.experimental.pallas import tpu as pltpu
from jax.experimental.pallas import tpu_sc as plsc

VOCAB = 1000
CHAR_DIM = 40
GAZET_DIM = 15
CTX = 21
SEQ = 16384
EMBED = CHAR_DIM + GAZET_DIM

NC, NS = 2, 16
NW = NC * NS
S_W = SEQ // NW
N_SL = S_W // 16


def _sc_body(tab_hbm, sent_hbm, gaz_hbm, out_hbm,
             tab_v, idx_v, gaz0, gaz1, out0, out1, sem_g, sem_o):
    wid = lax.axis_index("s") * NC + lax.axis_index("c")
    s0 = wid * S_W
    gazs = (gaz0, gaz1)
    outs = (out0, out1)

    pltpu.sync_copy(tab_hbm, tab_v)
    pltpu.sync_copy(sent_hbm.at[:, pl.ds(s0, S_W)], idx_v)
    pltpu.async_copy(gaz_hbm.at[0, :, pl.ds(s0, S_W)], gaz0, sem_g)

    def compute(c, gaz_v, out_v):
        jc = (c.astype(jnp.float32) + 1.0) * (1.0 / CTX)

        @plsc.parallel_loop(0, N_SL)
        def chunk(j):
            idx = idx_v[c, pl.ds(j * 16, 16)]
            for e in range(CHAR_DIM):
                ke = float(e + 1) / EMBED
                pe_ce = (1.0 - ke) - jc * (1.0 - 2.0 * ke)
                vals = plsc.load_gather(tab_v, [idx + e * VOCAB])
                out_v[e, pl.ds(j * 16, 16)] = vals + pe_ce
            for g in range(GAZET_DIM):
                e = CHAR_DIM + g
                ke = float(e + 1) / EMBED
                pe_ce = (1.0 - ke) - jc * (1.0 - 2.0 * ke)
                gv = gaz_v[g, pl.ds(j * 16, 16)]
                out_v[e, pl.ds(j * 16, 16)] = gv + pe_ce

    def pair(cc, carry):
        for b in range(2):
            c = cc * 2 + b

            @pl.when(c < CTX)
            def _():
                pltpu.make_async_copy(
                    gaz_hbm.at[0, :, pl.ds(s0, S_W)], gazs[b], sem_g
                ).wait()

                @pl.when(c + 1 < CTX)
                def _():
                    pltpu.async_copy(
                        gaz_hbm.at[c + 1, :, pl.ds(s0, S_W)], gazs[1 - b], sem_g
                    )

                @pl.when(c >= 2)
                def _():
                    pltpu.make_async_copy(
                        outs[b], out_hbm.at[0, :, pl.ds(s0, S_W)], sem_o
                    ).wait()

                compute(c, gazs[b], outs[b])
                pltpu.async_copy(
                    outs[b], out_hbm.at[c, :, pl.ds(s0, S_W)], sem_o
                )
        return carry

    lax.fori_loop(0, (CTX + 1) // 2, pair, 0)

    for b in range(2):
        pltpu.make_async_copy(
            outs[b], out_hbm.at[0, :, pl.ds(s0, S_W)], sem_o
        ).wait()


@jax.jit
def _run(tab_flat, sent_t, gaz_t):
    mesh = plsc.VectorSubcoreMesh(core_axis_name="c", subcore_axis_name="s")
    k = pl.kernel(
        _sc_body,
        out_type=jax.ShapeDtypeStruct((CTX, EMBED, SEQ), jnp.float32),
        mesh=mesh,
        scratch_types=[
            pltpu.VMEM((CHAR_DIM * VOCAB,), jnp.float32),
            pltpu.VMEM((CTX, S_W), jnp.int32),
            pltpu.VMEM((GAZET_DIM, S_W), jnp.float32),
            pltpu.VMEM((GAZET_DIM, S_W), jnp.float32),
            pltpu.VMEM((EMBED, S_W), jnp.float32),
            pltpu.VMEM((EMBED, S_W), jnp.float32),
            pltpu.SemaphoreType.DMA,
            pltpu.SemaphoreType.DMA,
        ],
        compiler_params=pltpu.CompilerParams(
            needs_layout_passes=False, use_tc_tiling_on_sc=False
        ),
    )
    return k(tab_flat, sent_t, gaz_t)


def kernel(sentence, gazet, char_table):
    tab_flat = char_table.T.reshape(CHAR_DIM * VOCAB)
    sent_t = jnp.swapaxes(sentence, 0, 1).astype(jnp.int32)
    gaz_t = jnp.transpose(gazet, (1, 2, 0))
    out = _run(tab_flat, sent_t, gaz_t)
    return jnp.transpose(out, (2, 0, 1))


# speedup vs baseline: 7.8959x; 1.2746x over previous
import jax
import jax.numpy as jnp
from jax import lax
from jax.experimental import pallas as pl
from jax.experimental.pallas import tpu as pltpu
from jax.experimental.pallas import tpu_sc as plsc

VOCAB = 1000
CHAR_DIM = 40
GAZET_DIM = 15
CTX = 21
SEQ = 16384
EMBED = CHAR_DIM + GAZET_DIM

NC, NS = 2, 16
NW = NC * NS
S_W = SEQ // NW
N_SL = S_W // 16


def _sc_body(tab_hbm, sent_hbm, gaz_hbm, out_hbm,
             tab_v, idx_v, gaz0, gaz1, out0, out1, sem_g, sem_o):
    wid = lax.axis_index("s") * NC + lax.axis_index("c")
    s0 = wid * S_W
    gazs = (gaz0, gaz1)
    outs = (out0, out1)

    pltpu.sync_copy(tab_hbm, tab_v)
    pltpu.sync_copy(sent_hbm.at[:, pl.ds(s0, S_W)], idx_v)
    pltpu.async_copy(gaz_hbm.at[0, :, pl.ds(s0, S_W)], gaz0, sem_g)

    def compute(c, gaz_v, out_v):
        jc = (c.astype(jnp.float32) + 1.0) * (1.0 / CTX)

        @plsc.parallel_loop(0, N_SL, unroll=2)
        def chunk(j):
            idx = idx_v[c, pl.ds(j * 16, 16)]
            for e in range(CHAR_DIM):
                ke = float(e + 1) / EMBED
                pe_ce = (1.0 - ke) - jc * (1.0 - 2.0 * ke)
                vals = plsc.load_gather(tab_v, [idx + e * VOCAB])
                out_v[e, pl.ds(j * 16, 16)] = vals + pe_ce

        @plsc.parallel_loop(0, N_SL, unroll=2)
        def chunk_g(j):
            for g in range(GAZET_DIM):
                e = CHAR_DIM + g
                ke = float(e + 1) / EMBED
                pe_ce = (1.0 - ke) - jc * (1.0 - 2.0 * ke)
                gv = gaz_v[g, pl.ds(j * 16, 16)]
                out_v[e, pl.ds(j * 16, 16)] = gv + pe_ce

    def pair(cc, carry):
        for b in range(2):
            c = cc * 2 + b

            @pl.when(c < CTX)
            def _():
                pltpu.make_async_copy(
                    gaz_hbm.at[0, :, pl.ds(s0, S_W)], gazs[b], sem_g
                ).wait()

                @pl.when(c + 1 < CTX)
                def _():
                    pltpu.async_copy(
                        gaz_hbm.at[c + 1, :, pl.ds(s0, S_W)], gazs[1 - b], sem_g
                    )

                @pl.when(c >= 2)
                def _():
                    pltpu.make_async_copy(
                        outs[b], out_hbm.at[0, :, pl.ds(s0, S_W)], sem_o
                    ).wait()

                compute(c, gazs[b], outs[b])
                pltpu.async_copy(
                    outs[b], out_hbm.at[c, :, pl.ds(s0, S_W)], sem_o
                )
        return carry

    lax.fori_loop(0, (CTX + 1) // 2, pair, 0)

    for b in range(2):
        pltpu.make_async_copy(
            outs[b], out_hbm.at[0, :, pl.ds(s0, S_W)], sem_o
        ).wait()


@jax.jit
def _run(tab_flat, sent_t, gaz_t):
    mesh = plsc.VectorSubcoreMesh(core_axis_name="c", subcore_axis_name="s")
    k = pl.kernel(
        _sc_body,
        out_type=jax.ShapeDtypeStruct((CTX, EMBED, SEQ), jnp.float32),
        mesh=mesh,
        scratch_types=[
            pltpu.VMEM((CHAR_DIM * VOCAB,), jnp.float32),
            pltpu.VMEM((CTX, S_W), jnp.int32),
            pltpu.VMEM((GAZET_DIM, S_W), jnp.float32),
            pltpu.VMEM((GAZET_DIM, S_W), jnp.float32),
            pltpu.VMEM((EMBED, S_W), jnp.float32),
            pltpu.VMEM((EMBED, S_W), jnp.float32),
            pltpu.SemaphoreType.DMA,
            pltpu.SemaphoreType.DMA,
        ],
        compiler_params=pltpu.CompilerParams(
            needs_layout_passes=False, use_tc_tiling_on_sc=False
        ),
    )
    return k(tab_flat, sent_t, gaz_t)


def kernel(sentence, gazet, char_table):
    tab_flat = char_table.T.reshape(CHAR_DIM * VOCAB)
    sent_t = jnp.swapaxes(sentence, 0, 1).astype(jnp.int32)
    gaz_t = jnp.transpose(gazet, (1, 2, 0))
    out = _run(tab_flat, sent_t, gaz_t)
    return jnp.transpose(out, (2, 0, 1))
